# baseline (speedup 1.0000x reference)
import functools

import jax
import jax.numpy as jnp
from jax import lax
from jax.experimental import pallas as pl
from jax.experimental.pallas import tpu as pltpu
from jax.experimental.pallas import tpu_sc as plsc

NUM_USERS = 20000
NUM_ITEMS = 30000
N_NODES = NUM_USERS + NUM_ITEMS
D = 64
DH = D // 2
N_LAYERS = 3

CHUNK = 128
BLK = 3
EPB = BLK * CHUNK
ROWS_PER_TILE = N_NODES // 16


def _prop_body(tables, esrc, edst, ewgt, zrows, out1, out2, out3,
               sb, db, wb, rows, acc, semi, sem, sem_s):
    cid = lax.axis_index("c")
    sid = lax.axis_index("s")

    ep_tile = esrc.shape[0] // 16
    n_blocks = ep_tile // EPB
    n_half = n_blocks // 2
    e0 = sid * ep_tile

    def load_block(b, q):
        for j in range(BLK):
            off = e0 + b * EPB + j * CHUNK
            pltpu.async_copy(esrc.at[pl.ds(off, CHUNK)], sb.at[q, j], semi)
            pltpu.async_copy(edst.at[pl.ds(off, CHUNK)], db.at[q, j], semi)
            pltpu.async_copy(ewgt.at[pl.ds(off, CHUNK)], wb.at[q, j], semi)
        for j in range(BLK):
            off = e0 + b * EPB + j * CHUNK
            pltpu.make_async_copy(esrc.at[pl.ds(off, CHUNK)], sb.at[q, j], semi).wait()
            pltpu.make_async_copy(edst.at[pl.ds(off, CHUNK)], db.at[q, j], semi).wait()
            pltpu.make_async_copy(ewgt.at[pl.ds(off, CHUNK)], wb.at[q, j], semi).wait()

    def issue_gather(tab, q, p):
        for j in range(BLK):
            pltpu.async_copy(tab.at[sb.at[q, j]],
                             rows.at[p, pl.ds(j * CHUNK, CHUNK)], sem.at[p])

    def wait_gather(tab, q, p):
        for j in range(BLK):
            pltpu.make_async_copy(tab.at[sb.at[q, j]],
                                  rows.at[p, pl.ds(j * CHUNK, CHUNK)],
                                  sem.at[p]).wait()

    def scale(q):
        for g in range(EPB // 16):
            j, i = divmod(g, CHUNK // 16)
            w16 = wb[q, j, pl.ds(i * 16, 16)]
            r0 = g * 16
            for e in range(16):
                we = w16[e]
                for k in range(DH // 16):
                    rows[q, r0 + e, pl.ds(k * 16, 16)] = (
                        rows[q, r0 + e, pl.ds(k * 16, 16)] * we)

    def issue_scatter(q):
        for j in range(BLK):
            pltpu.async_copy(rows.at[q, pl.ds(j * CHUNK, CHUNK)],
                             acc.at[db.at[q, j]], sem_s.at[q], add=True)

    def wait_scatter(q):
        for j in range(BLK):
            pltpu.make_async_copy(rows.at[q, pl.ds(j * CHUNK, CHUNK)],
                                  acc.at[db.at[q, j]], sem_s.at[q]).wait()

    def run_layer(tab, out):
        pltpu.sync_copy(zrows,
                        acc.at[pl.ds(sid * ROWS_PER_TILE, ROWS_PER_TILE)])
        plsc.subcore_barrier()

        load_block(0, 0)
        issue_gather(tab, 0, 0)

        def loop_body(t, carry):
            b0 = 2 * t
            @pl.when(t >= 1)
            def _():
                wait_scatter(1)
            load_block(b0 + 1, 1)
            issue_gather(tab, 1, 1)
            wait_gather(tab, 0, 0)
            scale(0)
            issue_scatter(0)

            @pl.when(t + 1 < n_half)
            def _():
                wait_scatter(0)
                load_block(b0 + 2, 0)
                issue_gather(tab, 0, 0)
            wait_gather(tab, 1, 1)
            scale(1)
            issue_scatter(1)
            return carry

        lax.fori_loop(0, n_half, loop_body, 0)
        wait_scatter(0)
        wait_scatter(1)
        plsc.subcore_barrier()
        r0 = sid * ROWS_PER_TILE
        pltpu.sync_copy(acc.at[pl.ds(r0, ROWS_PER_TILE)],
                        out.at[cid, pl.ds(r0, ROWS_PER_TILE)])
        plsc.subcore_barrier()

    run_layer(tables.at[cid], out1)
    run_layer(out1.at[cid], out2)
    run_layer(out2.at[cid], out3)


def _propagate_all(tables, esrc, edst, ewgt, zrows):
    mesh = plsc.VectorSubcoreMesh(core_axis_name="c", subcore_axis_name="s")
    t = jax.ShapeDtypeStruct((2, N_NODES, DH), jnp.float32)
    return pl.kernel(
        _prop_body,
        out_type=[t, t, t],
        mesh=mesh,
        compiler_params=pltpu.CompilerParams(use_tc_tiling_on_sc=False),
        scratch_types=[
            pltpu.VMEM((2, BLK, CHUNK), jnp.int32),
            pltpu.VMEM((2, BLK, CHUNK), jnp.int32),
            pltpu.VMEM((2, BLK, CHUNK), jnp.float32),
            pltpu.VMEM((2, EPB, DH), jnp.float32),
            pltpu.VMEM_SHARED((N_NODES, DH), jnp.float32),
            pltpu.SemaphoreType.DMA,
            pltpu.SemaphoreType.DMA((2,)),
            pltpu.SemaphoreType.DMA((2,)),
        ],
    )(tables, esrc, edst, ewgt, zrows)


def _mean4_body(a, b, c, d, o):
    o[...] = (a[...] + b[...] + c[...] + d[...]) * 0.25


def _mean4(t0, t1, t2, t3):
    flat = 2 * N_NODES * DH // 128
    blk = flat // 25
    args = [t.reshape(flat, 128) for t in (t0, t1, t2, t3)]
    spec = pl.BlockSpec((blk, 128), lambda i: (i, 0))
    out = pl.pallas_call(
        _mean4_body,
        grid=(25,),
        in_specs=[spec] * 4,
        out_specs=spec,
        out_shape=jax.ShapeDtypeStruct((flat, 128), jnp.float32),
    )(*args)
    return out.reshape(2, N_NODES, DH)


@functools.partial(jax.jit)
def kernel(user_emb, item_emb, edge_index, edge_weight):
    full = jnp.concatenate([user_emb, item_emb], axis=0)
    tables = jnp.stack([full[:, :DH], full[:, DH:]], axis=0)

    n_edges = edge_index.shape[1]
    ep_tile = -(-n_edges // (16 * 2 * EPB)) * 2 * EPB

    epad = 16 * ep_tile
    pad = epad - n_edges
    src = jnp.concatenate([edge_index[0], jnp.zeros((pad,), jnp.int32)])
    dst = jnp.concatenate([edge_index[1], jnp.zeros((pad,), jnp.int32)])
    wgt = jnp.concatenate([edge_weight, jnp.zeros((pad,), jnp.float32)])

    zrows = jnp.zeros((ROWS_PER_TILE, DH), jnp.float32)

    e1, e2, e3 = _propagate_all(tables, src, dst, wgt, zrows)
    light_out = _mean4(tables, e1, e2, e3)
    full_out = jnp.concatenate([light_out[0], light_out[1]], axis=1)
    return full_out[:NUM_USERS], full_out[NUM_USERS:]


# speedup vs baseline: 1.2362x; 1.2362x over previous
import functools

import jax
import jax.numpy as jnp
from jax import lax
from jax.experimental import pallas as pl
from jax.experimental.pallas import tpu as pltpu
from jax.experimental.pallas import tpu_sc as plsc

NUM_USERS = 20000
NUM_ITEMS = 30000
N_NODES = NUM_USERS + NUM_ITEMS
D = 64
DH = D // 2
N_LAYERS = 3

CHUNK = 128
BLK = 2
EPB = BLK * CHUNK
ROWS_PER_TILE = N_NODES // 16


def _prop_body(tables, esrc, edst, ewgt, zrows, out1, out2, out3,
               sb, db, wb, rows, acc, semi, sem, sem_s):
    cid = lax.axis_index("c")
    sid = lax.axis_index("s")

    ep_tile = esrc.shape[0] // 16
    n_blocks = ep_tile // EPB
    n_half = n_blocks // 2
    e0 = sid * ep_tile

    def load_block(b, q):
        for j in range(BLK):
            off = e0 + b * EPB + j * CHUNK
            pltpu.async_copy(esrc.at[pl.ds(off, CHUNK)], sb.at[q, j], semi)
            pltpu.async_copy(edst.at[pl.ds(off, CHUNK)], db.at[q, j], semi)
            pltpu.async_copy(ewgt.at[pl.ds(off, CHUNK)], wb.at[q, j], semi)
        for j in range(BLK):
            off = e0 + b * EPB + j * CHUNK
            pltpu.make_async_copy(esrc.at[pl.ds(off, CHUNK)], sb.at[q, j], semi).wait()
            pltpu.make_async_copy(edst.at[pl.ds(off, CHUNK)], db.at[q, j], semi).wait()
            pltpu.make_async_copy(ewgt.at[pl.ds(off, CHUNK)], wb.at[q, j], semi).wait()

    def issue_gather(tab, q, p):
        for j in range(BLK):
            pltpu.async_copy(tab.at[sb.at[q, j]],
                             rows.at[p, pl.ds(j * CHUNK, CHUNK)], sem.at[p])

    def wait_gather(tab, q, p):
        for j in range(BLK):
            pltpu.make_async_copy(tab.at[sb.at[q, j]],
                                  rows.at[p, pl.ds(j * CHUNK, CHUNK)],
                                  sem.at[p]).wait()

    def scale(q):
        for g in range(EPB // 16):
            j, i = divmod(g, CHUNK // 16)
            w16 = wb[q, j, pl.ds(i * 16, 16)]
            r0 = g * 16
            for e in range(16):
                we = w16[e]
                for k in range(DH // 16):
                    rows[q, r0 + e, pl.ds(k * 16, 16)] = (
                        rows[q, r0 + e, pl.ds(k * 16, 16)] * we)

    def issue_scatter(q):
        for j in range(BLK):
            pltpu.async_copy(rows.at[q, pl.ds(j * CHUNK, CHUNK)],
                             acc.at[db.at[q, j]], sem_s.at[q], add=True)

    def wait_scatter(q):
        for j in range(BLK):
            pltpu.make_async_copy(rows.at[q, pl.ds(j * CHUNK, CHUNK)],
                                  acc.at[db.at[q, j]], sem_s.at[q]).wait()

    def run_layer(tab, out):
        pltpu.sync_copy(zrows,
                        acc.at[pl.ds(sid * ROWS_PER_TILE, ROWS_PER_TILE)])
        plsc.subcore_barrier()

        load_block(0, 0)
        issue_gather(tab, 0, 0)

        def loop_body(t, carry):
            b0 = 2 * t
            @pl.when(t >= 1)
            def _():
                wait_scatter(1)
            load_block(b0 + 1, 1)
            issue_gather(tab, 1, 1)
            wait_gather(tab, 0, 0)
            scale(0)
            issue_scatter(0)

            @pl.when(t + 1 < n_half)
            def _():
                wait_scatter(0)
                load_block(b0 + 2, 0)
                issue_gather(tab, 0, 0)
            wait_gather(tab, 1, 1)
            scale(1)
            issue_scatter(1)
            return carry

        lax.fori_loop(0, n_half, loop_body, 0)
        wait_scatter(0)
        wait_scatter(1)
        plsc.subcore_barrier()
        r0 = sid * ROWS_PER_TILE
        pltpu.sync_copy(acc.at[pl.ds(r0, ROWS_PER_TILE)],
                        out.at[cid, pl.ds(r0, ROWS_PER_TILE)])
        plsc.subcore_barrier()

    run_layer(tables.at[cid], out1)
    run_layer(out1.at[cid], out2)
    run_layer(out2.at[cid], out3)


def _propagate_all(tables, esrc, edst, ewgt, zrows):
    mesh = plsc.VectorSubcoreMesh(core_axis_name="c", subcore_axis_name="s")
    t = jax.ShapeDtypeStruct((2, N_NODES, DH), jnp.float32)
    return pl.kernel(
        _prop_body,
        out_type=[t, t, t],
        mesh=mesh,
        compiler_params=pltpu.CompilerParams(use_tc_tiling_on_sc=False),
        scratch_types=[
            pltpu.VMEM((2, BLK, CHUNK), jnp.int32),
            pltpu.VMEM((2, BLK, CHUNK), jnp.int32),
            pltpu.VMEM((2, BLK, CHUNK), jnp.float32),
            pltpu.VMEM((2, EPB, DH), jnp.float32),
            pltpu.VMEM_SHARED((N_NODES, DH), jnp.float32),
            pltpu.SemaphoreType.DMA,
            pltpu.SemaphoreType.DMA((2,)),
            pltpu.SemaphoreType.DMA((2,)),
        ],
    )(tables, esrc, edst, ewgt, zrows)


def _mean4_body(a, b, c, d, o):
    o[...] = (a[...] + b[...] + c[...] + d[...]) * 0.25


def _mean4(t0, t1, t2, t3):
    flat = 2 * N_NODES * DH // 128
    blk = flat // 25
    args = [t.reshape(flat, 128) for t in (t0, t1, t2, t3)]
    spec = pl.BlockSpec((blk, 128), lambda i: (i, 0))
    out = pl.pallas_call(
        _mean4_body,
        grid=(25,),
        in_specs=[spec] * 4,
        out_specs=spec,
        out_shape=jax.ShapeDtypeStruct((flat, 128), jnp.float32),
    )(*args)
    return out.reshape(2, N_NODES, DH)


@functools.partial(jax.jit)
def kernel(user_emb, item_emb, edge_index, edge_weight):
    full = jnp.concatenate([user_emb, item_emb], axis=0)
    tables = jnp.stack([full[:, :DH], full[:, DH:]], axis=0)

    n_edges = edge_index.shape[1]
    ep_tile = -(-n_edges // (16 * 2 * EPB)) * 2 * EPB

    epad = 16 * ep_tile
    pad = epad - n_edges
    src = jnp.concatenate([edge_index[0], jnp.zeros((pad,), jnp.int32)])
    dst = jnp.concatenate([edge_index[1], jnp.zeros((pad,), jnp.int32)])
    wgt = jnp.concatenate([edge_weight, jnp.zeros((pad,), jnp.float32)])

    zrows = jnp.zeros((ROWS_PER_TILE, DH), jnp.float32)

    e1, e2, e3 = _propagate_all(tables, src, dst, wgt, zrows)
    light_out = _mean4(tables, e1, e2, e3)
    full_out = jnp.concatenate([light_out[0], light_out[1]], axis=1)
    return full_out[:NUM_USERS], full_out[NUM_USERS:]
